# baseline (speedup 1.0000x reference)
import jax
import jax.numpy as jnp
from jax import lax
from jax.experimental import pallas as pl
from jax.experimental.pallas import tpu as pltpu

FILL = 64
BM = 512
NJ = 2


def _linear_kernel(x_ref, w_ref, b_ref, o_ref, ws_ref):
    t = pl.program_id(0)
    rs = w_ref.shape[0]
    nh = o_ref.shape[1]

    @pl.when(t < FILL)
    def _fill():
        ws_ref[pl.ds(t * rs, rs), :] = w_ref[...].astype(jnp.bfloat16)

    @pl.when(t >= FILL)
    def _compute():
        j = (t - FILL) % NJ
        xb = x_ref[...].astype(jnp.bfloat16)
        o_ref[...] = lax.dot_general(
            xb, ws_ref[pl.ds(j * nh, nh), :], (((1,), (1,)), ((), ())),
            preferred_element_type=jnp.float32,
        ) + b_ref[...]


def kernel(input, weight, bias):
    m, kdim = input.shape
    n, _ = weight.shape
    bias2 = bias.reshape(1, n)
    nsteps = FILL + (m // BM) * NJ
    return pl.pallas_call(
        _linear_kernel,
        grid=(nsteps,),
        in_specs=[
            pl.BlockSpec(
                (BM, kdim), lambda t: (jnp.maximum(t - FILL, 0) // NJ, 0)
            ),
            pl.BlockSpec(
                (n // FILL, kdim), lambda t: (jnp.minimum(t, FILL - 1), 0)
            ),
            pl.BlockSpec((1, n // NJ), lambda t: (0, jnp.maximum(t - FILL, 0) % NJ)),
        ],
        out_specs=pl.BlockSpec(
            (BM, n // NJ),
            lambda t: (jnp.maximum(t - FILL, 0) // NJ, jnp.maximum(t - FILL, 0) % NJ),
        ),
        out_shape=jax.ShapeDtypeStruct((m, n), jnp.float32),
        scratch_shapes=[pltpu.VMEM((n, kdim), jnp.bfloat16)],
        compiler_params=pltpu.CompilerParams(
            dimension_semantics=("arbitrary",),
            vmem_limit_bytes=64 * 1024 * 1024,
        ),
    )(input, weight, bias2)


# speedup vs baseline: 1.0897x; 1.0897x over previous
import jax
import jax.numpy as jnp
from jax import lax
from jax.experimental import pallas as pl
from jax.experimental.pallas import tpu as pltpu

FILL = 32
BM = 512
NJ = 2


def _linear_kernel(x_ref, w_ref, b_ref, o_ref, ws_ref):
    t = pl.program_id(0)
    rs = w_ref.shape[0]
    nh = o_ref.shape[1]

    @pl.when(t < FILL)
    def _fill():
        ws_ref[pl.ds(t * rs, rs), :] = w_ref[...].astype(jnp.bfloat16)

    @pl.when(t >= FILL)
    def _compute():
        j = (t - FILL) % NJ
        xb = x_ref[...].astype(jnp.bfloat16)
        o_ref[...] = lax.dot_general(
            xb, ws_ref[pl.ds(j * nh, nh), :], (((1,), (1,)), ((), ())),
            preferred_element_type=jnp.float32,
        ) + b_ref[...]


def kernel(input, weight, bias):
    m, kdim = input.shape
    n, _ = weight.shape
    bias2 = bias.reshape(1, n)
    nsteps = FILL + (m // BM) * NJ
    return pl.pallas_call(
        _linear_kernel,
        grid=(nsteps,),
        in_specs=[
            pl.BlockSpec(
                (BM, kdim), lambda t: (jnp.maximum(t - FILL, 0) // NJ, 0)
            ),
            pl.BlockSpec(
                (n // FILL, kdim), lambda t: (jnp.minimum(t, FILL - 1), 0)
            ),
            pl.BlockSpec((1, n // NJ), lambda t: (0, jnp.maximum(t - FILL, 0) % NJ)),
        ],
        out_specs=pl.BlockSpec(
            (BM, n // NJ),
            lambda t: (jnp.maximum(t - FILL, 0) // NJ, jnp.maximum(t - FILL, 0) % NJ),
        ),
        out_shape=jax.ShapeDtypeStruct((m, n), jnp.float32),
        scratch_shapes=[pltpu.VMEM((n, kdim), jnp.bfloat16)],
        compiler_params=pltpu.CompilerParams(
            dimension_semantics=("arbitrary",),
            vmem_limit_bytes=64 * 1024 * 1024,
        ),
    )(input, weight, bias2)
